# baseline (speedup 1.0000x reference)
import jax
import jax.numpy as jnp
from jax import lax
from jax.experimental import pallas as pl
from jax.experimental.pallas import tpu as pltpu
from jax.experimental.pallas import tpu_sc as plsc

D = 768
DV = D // 16
LN_EPS = 1e-12
NW = 32


def _rsqrt_vec(v):
    i = plsc.bitcast(v, jnp.int32)
    i = jnp.int32(0x5F3759DF) - lax.shift_right_arithmetic(i, jnp.int32(1))
    y = plsc.bitcast(i, jnp.float32)
    half = v * jnp.float32(0.5)
    for _ in range(3):
        y = y * (jnp.float32(1.5) - half * y * y)
    return y


def _make_emb_ln(n_rows, chunk):
    per_w = n_rows // NW
    ng = per_w // chunk
    assert per_w % chunk == 0 and n_rows % NW == 0

    mesh = plsc.VectorSubcoreMesh(
        core_axis_name="c", subcore_axis_name="s", num_cores=2, num_subcores=16
    )

    def body(ids_hbm, table_hbm, seg_hbm, gamma_hbm, beta_hbm, out_hbm,
             idx_v, rows_v, seg_v, gamma_v, beta_v, gsem, wsem):
        wid = lax.axis_index("s") * 2 + lax.axis_index("c")
        base = wid * per_w
        pltpu.sync_copy(ids_hbm.at[pl.ds(base, per_w)], idx_v)
        pltpu.sync_copy(seg_hbm, seg_v)
        pltpu.sync_copy(gamma_hbm, gamma_v)
        pltpu.sync_copy(beta_hbm, beta_v)

        def chunk_body(g, _):
            pltpu.async_copy(
                table_hbm.at[idx_v.at[pl.ds(g * chunk, chunk)]], rows_v, gsem
            ).wait()

            def row_body(r, _):
                def p1(j, carry):
                    a1, a2 = carry
                    sl = pl.ds(j * 16, 16)
                    y = rows_v[r, sl] + seg_v[sl]
                    rows_v[r, sl] = y
                    return a1 + y, a2 + y * y

                a1, a2 = lax.fori_loop(
                    0, DV, p1,
                    (jnp.zeros(16, jnp.float32), jnp.zeros(16, jnp.float32)),
                )
                mean = jnp.sum(a1) * jnp.float32(1.0 / D)
                var = jnp.sum(a2) * jnp.float32(1.0 / D) - mean * mean
                inv = _rsqrt_vec(lax.broadcast(var + jnp.float32(LN_EPS), (16,)))
                mv = lax.broadcast(mean, (16,))

                def p2(j, _):
                    sl = pl.ds(j * 16, 16)
                    rows_v[r, sl] = (rows_v[r, sl] - mv) * inv * gamma_v[sl] + beta_v[sl]
                    return 0

                lax.fori_loop(0, DV, p2, 0)
                return 0

            lax.fori_loop(0, chunk, row_body, 0)
            pltpu.async_copy(
                rows_v, out_hbm.at[pl.ds(base + g * chunk, chunk)], wsem
            ).wait()
            return 0

        lax.fori_loop(0, ng, chunk_body, 0)

    return pl.kernel(
        body,
        out_type=jax.ShapeDtypeStruct((n_rows, D), jnp.float32),
        mesh=mesh,
        compiler_params=pltpu.CompilerParams(needs_layout_passes=False),
        scratch_types=[
            pltpu.VMEM((per_w,), jnp.int32),
            pltpu.VMEM((chunk, D), jnp.float32),
            pltpu.VMEM((D,), jnp.float32),
            pltpu.VMEM((D,), jnp.float32),
            pltpu.VMEM((D,), jnp.float32),
            pltpu.SemaphoreType.DMA,
            pltpu.SemaphoreType.DMA,
        ],
    )


def kernel(input_ids, word_table, segment_table, ln_gamma, ln_beta):
    b, s = input_ids.shape
    n = b * s
    ids = input_ids.reshape(n).astype(jnp.int32)
    out = _make_emb_ln(n, 64)(
        ids, word_table, segment_table[0], ln_gamma, ln_beta
    )
    return out.reshape(b, s, D), jnp.zeros_like(input_ids)


# speedup vs baseline: 1.9731x; 1.9731x over previous
import jax
import jax.numpy as jnp
from jax import lax
from jax.experimental import pallas as pl
from jax.experimental.pallas import tpu as pltpu
from jax.experimental.pallas import tpu_sc as plsc

D = 768
DV = D // 16
LN_EPS = 1e-12
NW = 32
CHUNK = 32
RB = 4
UNROLL = 8


def _rsqrt_scalar_to_vec(var):
    v = lax.broadcast(var, (16,))
    i = plsc.bitcast(v, jnp.int32)
    i = jnp.int32(0x5F3759DF) - lax.shift_right_arithmetic(i, jnp.int32(1))
    y = plsc.bitcast(i, jnp.float32)
    half = v * jnp.float32(0.5)
    for _ in range(3):
        y = y * (jnp.float32(1.5) - half * y * y)
    return y


def _make_emb_ln(n_rows):
    per_w = n_rows // NW
    ng = per_w // CHUNK
    assert per_w % CHUNK == 0 and n_rows % NW == 0 and ng % 2 == 0

    mesh = plsc.VectorSubcoreMesh(
        core_axis_name="c", subcore_axis_name="s", num_cores=2, num_subcores=16
    )

    def body(ids_hbm, table_hbm, seg_hbm, gamma_hbm, beta_hbm, out_hbm,
             idx_v, gbuf0, gbuf1, wbuf0, wbuf1, seg_v, gamma_v, beta_v,
             gs0, gs1, ws0, ws1):
        wid = lax.axis_index("s") * 2 + lax.axis_index("c")
        base = wid * per_w
        pltpu.sync_copy(ids_hbm.at[pl.ds(base, per_w)], idx_v)
        pltpu.sync_copy(seg_hbm, seg_v)
        pltpu.sync_copy(gamma_hbm, gamma_v)
        pltpu.sync_copy(beta_hbm, beta_v)

        gbufs = (gbuf0, gbuf1)
        wbufs = (wbuf0, wbuf1)
        gsems = (gs0, gs1)
        wsems = (ws0, ws1)

        def start_gather(g, b):
            pltpu.async_copy(
                table_hbm.at[idx_v.at[pl.ds(g * CHUNK, CHUNK)]],
                gbufs[b], gsems[b],
            )

        def start_writeback(g, b):
            pltpu.async_copy(
                wbufs[b], out_hbm.at[pl.ds(base + g * CHUNK, CHUNK)], wsems[b],
            )

        start_gather(0, 0)
        start_gather(1, 1)

        def compute_chunk(gb, wb):
            def block(bi, _):
                r0 = bi * RB

                def p1(j, carry):
                    accs = list(carry)
                    sl = pl.ds(j * 16, 16)
                    s = seg_v[sl]
                    for r in range(RB):
                        y = gb[r0 + r, sl] + s
                        wb[r0 + r, sl] = y
                        accs[2 * r] = accs[2 * r] + y
                        accs[2 * r + 1] = accs[2 * r + 1] + y * y
                    return tuple(accs)

                zero = jnp.zeros((16,), jnp.float32)
                accs = lax.fori_loop(0, DV, p1, (zero,) * (2 * RB),
                                     unroll=UNROLL)

                mvs, ivs = [], []
                for r in range(RB):
                    mean = jnp.sum(accs[2 * r]) * jnp.float32(1.0 / D)
                    var = (jnp.sum(accs[2 * r + 1]) * jnp.float32(1.0 / D)
                           - mean * mean)
                    ivs.append(_rsqrt_scalar_to_vec(var + jnp.float32(LN_EPS)))
                    mvs.append(lax.broadcast(mean, (16,)))

                def p2(j, _):
                    sl = pl.ds(j * 16, 16)
                    gj = gamma_v[sl]
                    bj = beta_v[sl]
                    for r in range(RB):
                        y = wb[r0 + r, sl]
                        wb[r0 + r, sl] = (y - mvs[r]) * ivs[r] * gj + bj
                    return 0

                lax.fori_loop(0, DV, p2, 0, unroll=UNROLL)
                return 0

            lax.fori_loop(0, CHUNK // RB, block, 0)

        def pair(i, _):
            for b in range(2):
                g = 2 * i + b
                pltpu.make_async_copy(
                    table_hbm.at[idx_v.at[pl.ds(g * CHUNK, CHUNK)]],
                    gbufs[b], gsems[b],
                ).wait()
                @pl.when(g >= 2)
                def _():
                    pltpu.make_async_copy(
                        wbufs[b],
                        out_hbm.at[pl.ds(base + (g - 2) * CHUNK, CHUNK)],
                        wsems[b],
                    ).wait()

                compute_chunk(gbufs[b], wbufs[b])
                start_writeback(g, b)

                @pl.when(g + 2 < ng)
                def _():
                    start_gather(g + 2, b)
            return 0

        lax.fori_loop(0, ng // 2, pair, 0)

        for b in range(2):
            pltpu.make_async_copy(
                wbufs[b],
                out_hbm.at[pl.ds(base + (ng - 2 + b) * CHUNK, CHUNK)],
                wsems[b],
            ).wait()

    return pl.kernel(
        body,
        out_type=jax.ShapeDtypeStruct((n_rows, D), jnp.float32),
        mesh=mesh,
        compiler_params=pltpu.CompilerParams(needs_layout_passes=False),
        scratch_types=[
            pltpu.VMEM((per_w,), jnp.int32),
            pltpu.VMEM((CHUNK, D), jnp.float32),
            pltpu.VMEM((CHUNK, D), jnp.float32),
            pltpu.VMEM((CHUNK, D), jnp.float32),
            pltpu.VMEM((CHUNK, D), jnp.float32),
            pltpu.VMEM((D,), jnp.float32),
            pltpu.VMEM((D,), jnp.float32),
            pltpu.VMEM((D,), jnp.float32),
            pltpu.SemaphoreType.DMA,
            pltpu.SemaphoreType.DMA,
            pltpu.SemaphoreType.DMA,
            pltpu.SemaphoreType.DMA,
        ],
    )


def kernel(input_ids, word_table, segment_table, ln_gamma, ln_beta):
    b, s = input_ids.shape
    n = b * s
    ids = input_ids.reshape(n).astype(jnp.int32)
    out = _make_emb_ln(n)(ids, word_table, segment_table[0], ln_gamma, ln_beta)
    return out.reshape(b, s, D), jnp.zeros_like(input_ids)
